# baseline (speedup 1.0000x reference)
import functools

import jax
import jax.numpy as jnp
from jax import lax
from jax.experimental import pallas as pl
from jax.experimental.pallas import tpu as pltpu
from jax.experimental.pallas import tpu_sc as plsc

B = 16384
ED = 64
TD = 32
NC = 2
NS = 16
NW = NC * NS
BPW = B // NW
CHUNK = 128
NCH = BPW // CHUNK

_mesh = plsc.VectorSubcoreMesh(core_axis_name="c", subcore_axis_name="s")


@functools.partial(
    pl.kernel,
    out_type=(
        jax.ShapeDtypeStruct((B // CHUNK, CHUNK, ED), jnp.float32),
        jax.ShapeDtypeStruct((B // CHUNK, CHUNK, TD), jnp.float32),
    ),
    mesh=_mesh,
    compiler_params=pltpu.CompilerParams(use_tc_tiling_on_sc=False),
    scratch_types=[
        pltpu.VMEM((NCH, CHUNK), jnp.int32),
        pltpu.VMEM((NCH, CHUNK), jnp.int32),
        pltpu.VMEM((NCH, CHUNK, ED), jnp.float32),
        pltpu.VMEM((NCH, CHUNK, TD), jnp.float32),
        pltpu.SemaphoreType.DMA,
        pltpu.SemaphoreType.DMA,
    ],
)
def _sc_gather(ent_hbm, typ_hbm, eidx_hbm, tidx_hbm, ent_out, typ_out,
               eidx_v, tidx_v, erows_v, trows_v, esem, tsem):
    wid = lax.axis_index("s") * NC + lax.axis_index("c")
    row0 = wid * NCH
    pltpu.sync_copy(eidx_hbm.at[pl.ds(row0, NCH)], eidx_v)
    pltpu.sync_copy(tidx_hbm.at[pl.ds(row0, NCH)], tidx_v)
    copies = []
    for j in range(NCH):
        copies.append(pltpu.async_copy(ent_hbm.at[eidx_v.at[j]], erows_v.at[j], esem))
        copies.append(pltpu.async_copy(typ_hbm.at[tidx_v.at[j]], trows_v.at[j], tsem))
    for c in copies:
        c.wait()
    pltpu.sync_copy(erows_v, ent_out.at[pl.ds(row0, NCH)])
    pltpu.sync_copy(trows_v, typ_out.at[pl.ds(row0, NCH)])


def _score_body(e_ref, t_ref, m_ref, g_ref, o_ref):
    s = jnp.dot(e_ref[...], m_ref[...], preferred_element_type=jnp.float32)
    s = s - t_ref[...]
    o_ref[...] = g_ref[0, 0] - jnp.sqrt(jnp.sum(s * s, axis=1, keepdims=True))


_score = pl.pallas_call(
    _score_body,
    out_shape=jax.ShapeDtypeStruct((B, 1), jnp.float32),
)


@jax.jit
def kernel(sample, entity_embedding, type_embedding, M, gamma):
    eidx = sample[:, 0].reshape(B // CHUNK, CHUNK)
    tidx = sample[:, 1].reshape(B // CHUNK, CHUNK)
    ent_rows, typ_rows = _sc_gather(entity_embedding, type_embedding, eidx, tidx)
    g = jnp.reshape(gamma, (1, 1)).astype(jnp.float32)
    return _score(ent_rows.reshape(B, ED), typ_rows.reshape(B, TD), M, g)
